# Initial kernel scaffold; baseline (speedup 1.0000x reference)
import jax
import jax.numpy as jnp
from jax.experimental import pallas as pl


def kernel(inputs, scale, mean, scale_table):
    raise NotImplementedError("write your pallas kernel here")


# speedup vs baseline: 232.6969x; 232.6969x over previous
import functools

import jax
import jax.numpy as jnp
from jax.experimental import pallas as pl
from jax.experimental.pallas import tpu as pltpu

_ROUND_C = float(1.5 * 2.0**23)


def _body(table_ref, x_ref, scale_ref, mean_ref, out_ref):
    t = [table_ref[k] for k in range(64)]
    s = jnp.abs(scale_ref[...])
    d_prev = s - t[0]
    qs = jnp.full_like(s, t[0])
    for k in range(1, 64):
        d = s - t[k]
        take = (d_prev + d) > 0.0
        qs = jnp.where(take, t[k], qs)
        d_prev = d
    v = (x_ref[...] - mean_ref[...]) / qs
    r = jnp.round(v)
    out_ref[...] = r * qs + mean_ref[...]


@jax.jit
def kernel(inputs, scale, mean, scale_table):
    B, H, W = inputs.shape
    n = B * H * W
    cols = 1024
    rows = n // cols
    blk = 128
    x2 = inputs.reshape(rows, cols)
    s2 = scale.reshape(rows, cols)
    m2 = mean.reshape(rows, cols)
    grid = (rows // blk,)
    bs = pl.BlockSpec((blk, cols), lambda i: (i, 0))
    out = pl.pallas_call(
        _body,
        grid=grid,
        in_specs=[
            pl.BlockSpec(memory_space=pltpu.SMEM),
            bs, bs, bs,
        ],
        out_specs=bs,
        out_shape=jax.ShapeDtypeStruct((rows, cols), jnp.float32),
    )(scale_table, x2, s2, m2)
    return out.reshape(B, H, W)
